# baseline (speedup 1.0000x reference)
import functools

import jax
import jax.numpy as jnp
from jax import lax
from jax.experimental import pallas as pl
from jax.experimental.pallas import tpu as pltpu
from jax.experimental.pallas import tpu_sc as plsc

N = 10000
H = 128
P_PAIRS = 65536

_BM = 200


def _dot(a, b):
    return lax.dot_general(
        a, b, (((1,), (0,)), ((), ())),
        precision=lax.Precision.DEFAULT,
        preferred_element_type=jnp.float32,
    )


def _prelude_body(x_ref, win_ref, bin_ref, whn_ref, h_ref, nb_ref):
    a = _dot(x_ref[...], win_ref[...]) + bin_ref[...]
    i = jax.nn.sigmoid(a[:, H:2 * H])
    ng = jnp.tanh(a[:, 2 * H:])
    h = (1.0 - i) * ng
    h_ref[...] = h
    nb_ref[...] = _dot(h, whn_ref[...])


def _gate_update(adj_ref, nb_ref, x_ref, h_ref, win_ref, bin_ref, whs_ref):
    g = _dot(adj_ref[...], nb_ref[...])
    a = _dot(x_ref[...], win_ref[...]) + bin_ref[...]
    s = _dot(h_ref[...], whs_ref[...])
    r = jax.nn.sigmoid(a[:, :H] + s[:, :H] + g[:, :H])
    i = jax.nn.sigmoid(a[:, H:2 * H] + s[:, H:2 * H] + g[:, H:2 * H])
    ng = jnp.tanh(a[:, 2 * H:] + r * (s[:, 2 * H:] + g[:, 2 * H:]))
    return (1.0 - i) * ng + i * h_ref[...]


def _step_body(adj_ref, nb_ref, x_ref, h_ref, win_ref, bin_ref, whs_ref,
               whn_ref, hout_ref, nbout_ref):
    hn = _gate_update(adj_ref, nb_ref, x_ref, h_ref, win_ref, bin_ref,
                      whs_ref)
    hout_ref[...] = hn
    nbout_ref[...] = _dot(hn, whn_ref[...])


def _final_body(adj_ref, nb_ref, x_ref, h_ref, win_ref, bin_ref, whs_ref,
                fc1w_ref, fc1b_ref, fc2w_ref, fc2b_ref, fc3w_ref, fc3b_ref,
                lng_ref, lnb_ref, d1x_ref, d1y_ref, d1b_ref,
                z_ref, zx_ref, zy_ref):
    hn = _gate_update(adj_ref, nb_ref, x_ref, h_ref, win_ref, bin_ref,
                      whs_ref)
    x = jnp.maximum(_dot(hn, fc1w_ref[...]) + fc1b_ref[...], 0.0)
    x = jnp.maximum(_dot(x, fc2w_ref[...]) + fc2b_ref[...], 0.0)
    x = jnp.maximum(_dot(x, fc3w_ref[...]) + fc3b_ref[...], 0.0)
    m = jnp.mean(x, axis=-1, keepdims=True)
    xc = x - m
    var = jnp.sum(xc * xc, axis=-1, keepdims=True) / (H - 1)
    z = lng_ref[...] * xc / (jnp.sqrt(var) + 1e-6) + lnb_ref[...]
    z_ref[...] = z
    zx_ref[...] = _dot(z, d1x_ref[...]) + d1b_ref[...]
    zy_ref[...] = _dot(z, d1y_ref[...])


def _row_spec(bm, cols):
    return pl.BlockSpec((bm, cols), lambda i: (i, 0))


def _full_spec(rows, cols):
    return pl.BlockSpec((rows, cols), lambda i: (0, 0))


_TC_PARAMS = pltpu.CompilerParams(
    dimension_semantics=("arbitrary",),
    vmem_limit_bytes=100 * 1024 * 1024,
)


def _run_prelude(inputs, win, bin_, whn):
    bm = 1000
    f = inputs.shape[1]
    return pl.pallas_call(
        _prelude_body,
        grid=(N // bm,),
        in_specs=[
            _row_spec(bm, f),
            _full_spec(f, 3 * H),
            _full_spec(1, 3 * H),
            _full_spec(H, 3 * H),
        ],
        out_specs=[
            _row_spec(bm, H),
            _row_spec(bm, 3 * H),
        ],
        out_shape=[
            jax.ShapeDtypeStruct((N, H), jnp.float32),
            jax.ShapeDtypeStruct((N, 3 * H), jnp.float32),
        ],
        compiler_params=_TC_PARAMS,
    )(inputs, win, bin_, whn)


def _run_step(adj, nb, inputs, h, win, bin_, whs, whn):
    return pl.pallas_call(
        _step_body,
        grid=(N // _BM,),
        in_specs=[
            _row_spec(_BM, N),
            _full_spec(N, 3 * H),
            _row_spec(_BM, H),
            _row_spec(_BM, H),
            _full_spec(H, 3 * H),
            _full_spec(1, 3 * H),
            _full_spec(H, 3 * H),
            _full_spec(H, 3 * H),
        ],
        out_specs=[
            _row_spec(_BM, H),
            _row_spec(_BM, 3 * H),
        ],
        out_shape=[
            jax.ShapeDtypeStruct((N, H), jnp.float32),
            jax.ShapeDtypeStruct((N, 3 * H), jnp.float32),
        ],
        compiler_params=_TC_PARAMS,
    )(adj, nb, inputs, h, win, bin_, whs, whn)


def _run_final(adj, nb, inputs, h, win, bin_, whs,
               fc1w, fc1b, fc2w, fc2b, fc3w, fc3b,
               lng, lnb, d1x, d1y, d1b):
    return pl.pallas_call(
        _final_body,
        grid=(N // _BM,),
        in_specs=[
            _row_spec(_BM, N),
            _full_spec(N, 3 * H),
            _row_spec(_BM, H),
            _row_spec(_BM, H),
            _full_spec(H, 3 * H),
            _full_spec(1, 3 * H),
            _full_spec(H, 3 * H),
            _full_spec(H, H), _full_spec(1, H),
            _full_spec(H, H), _full_spec(1, H),
            _full_spec(H, H), _full_spec(1, H),
            _full_spec(1, H), _full_spec(1, H),
            _full_spec(H, H), _full_spec(H, H), _full_spec(1, H),
        ],
        out_specs=[
            _row_spec(_BM, H),
            _row_spec(_BM, H),
            _row_spec(_BM, H),
        ],
        out_shape=[
            jax.ShapeDtypeStruct((N, H), jnp.float32),
            jax.ShapeDtypeStruct((N, H), jnp.float32),
            jax.ShapeDtypeStruct((N, H), jnp.float32),
        ],
        compiler_params=_TC_PARAMS,
    )(adj, nb, inputs, h, win, bin_, whs, fc1w, fc1b, fc2w, fc2b,
      fc3w, fc3b, lng, lnb, d1x, d1y, d1b)


_CHUNK = 128
_GROUPS = _CHUNK // 16


def _decode_body(zx_hbm, zy_hbm, xi_hbm, yi_hbm, d2_hbm, b2_hbm, out_hbm,
                 idxx_v, idxy_v, rxa, rya, rxb, ryb, d2_v, b2_v, out_v,
                 accbuf_v, semxa, semya, semxb, semyb):
    info = plsc.get_sparse_core_info()
    nc = info.num_cores
    wid = lax.axis_index("s") * nc + lax.axis_index("c")
    chunks = idxx_v.shape[0]

    pltpu.sync_copy(d2_hbm, d2_v)
    pltpu.sync_copy(b2_hbm, b2_v)
    pltpu.sync_copy(xi_hbm.at[pl.ds(wid * chunks, chunks)], idxx_v)
    pltpu.sync_copy(yi_hbm.at[pl.ds(wid * chunks, chunks)], idxy_v)

    lanes0 = lax.iota(jnp.int32, 16)

    def fire(ch, rx, ry, semx, semy):
        pltpu.async_copy(zx_hbm.at[idxx_v.at[ch]], rx, semx)
        pltpu.async_copy(zy_hbm.at[idxy_v.at[ch]], ry, semy)

    def wait(rx, ry, semx, semy):
        pltpu.make_async_copy(zx_hbm.at[idxx_v.at[0]], rx, semx).wait()
        pltpu.make_async_copy(zy_hbm.at[idxy_v.at[0]], ry, semy).wait()

    d2regs = [d2_v[pl.ds(16 * j, 16)] for j in range(H // 16)]

    def compute(ch, rx, ry):
        def group_body(g, _):
            for p in range(16):
                pr = g * 16 + p
                acc = jnp.zeros((16,), jnp.float32)
                for j in range(H // 16):
                    vx = rx[pr, pl.ds(16 * j, 16)]
                    vy = ry[pr, pl.ds(16 * j, 16)]
                    acc = acc + jnp.maximum(vx + vy, 0.0) * d2regs[j]
                accbuf_v[p, pl.ds(0, 16)] = acc
            tot = b2_v[...]
            for l in range(16):
                col = jnp.full((16,), l, jnp.int32)
                tot = tot + plsc.load_gather(accbuf_v, [lanes0, col])
            prob = 1.0 / (1.0 + jnp.exp(-tot))
            out_v[ch, pl.ds(g * 16, 16)] = prob
            return _

        lax.fori_loop(0, _GROUPS, group_body, 0, unroll=False)

    fire(0, rxa, rya, semxa, semya)

    def pair_body(i, _):
        fire(2 * i + 1, rxb, ryb, semxb, semyb)
        wait(rxa, rya, semxa, semya)
        compute(2 * i, rxa, rya)

        @pl.when(i < chunks // 2 - 1)
        def _fire_next():
            fire(2 * i + 2, rxa, rya, semxa, semya)

        wait(rxb, ryb, semxb, semyb)
        compute(2 * i + 1, rxb, ryb)
        return _

    lax.fori_loop(0, chunks // 2, pair_body, 0, unroll=False)
    pltpu.sync_copy(out_v, out_hbm.at[pl.ds(wid * chunks, chunks)])


def _run_decode(zx, zy, xi2d, yi2d, d2_vec, b2_vec):
    nw = 32
    chunks = P_PAIRS // (_CHUNK * nw)
    mesh = plsc.VectorSubcoreMesh(core_axis_name="c", subcore_axis_name="s")
    k = functools.partial(
        pl.kernel,
        out_type=jax.ShapeDtypeStruct((P_PAIRS // _CHUNK, _CHUNK), jnp.float32),
        mesh=mesh,
        compiler_params=pltpu.CompilerParams(needs_layout_passes=False),
        scratch_types=[
            pltpu.VMEM((chunks, _CHUNK), jnp.int32),
            pltpu.VMEM((chunks, _CHUNK), jnp.int32),
            pltpu.VMEM((_CHUNK, H), jnp.float32),
            pltpu.VMEM((_CHUNK, H), jnp.float32),
            pltpu.VMEM((_CHUNK, H), jnp.float32),
            pltpu.VMEM((_CHUNK, H), jnp.float32),
            pltpu.VMEM((H,), jnp.float32),
            pltpu.VMEM((16,), jnp.float32),
            pltpu.VMEM((chunks, _CHUNK), jnp.float32),
            pltpu.VMEM((16, 17), jnp.float32),
            pltpu.SemaphoreType.DMA,
            pltpu.SemaphoreType.DMA,
            pltpu.SemaphoreType.DMA,
            pltpu.SemaphoreType.DMA,
        ],
    )(_decode_body)
    return k(zx, zy, xi2d, yi2d, d2_vec, b2_vec)


def kernel(inputs, adj, x_idx, y_idx, ir_w, ir_b, ii_w, ii_b, in_w, in_b,
           hr_s, hr_n, hi_s, hi_n, hh_s, hh_n,
           fc1_w, fc1_b, fc2_w, fc2_b, fc3_w, fc3_b,
           ln_g, ln_b, d1_w, d1_b, d2_w, d2_b):
    win = jnp.concatenate([ir_w, ii_w, in_w], axis=1)
    bin_ = jnp.concatenate([ir_b, ii_b, in_b]).reshape(1, 3 * H)
    whs = jnp.concatenate([hr_s, hi_s, hh_s], axis=1)
    whn = jnp.concatenate([hr_n, hi_n, hh_n], axis=1)

    h1, nb1 = _run_prelude(inputs, win, bin_, whn)
    h2, nb2 = _run_step(adj, nb1, inputs, h1, win, bin_, whs, whn)
    z, zx, zy = _run_final(
        adj, nb2, inputs, h2, win, bin_, whs,
        fc1_w, fc1_b.reshape(1, H), fc2_w, fc2_b.reshape(1, H),
        fc3_w, fc3_b.reshape(1, H),
        ln_g.reshape(1, H), ln_b.reshape(1, H),
        d1_w[:H], d1_w[H:], d1_b.reshape(1, H))

    d2_vec = d2_w.reshape(H)
    b2_vec = jnp.full((16,), d2_b[0], jnp.float32)
    xi2d = x_idx.astype(jnp.int32).reshape(P_PAIRS // _CHUNK, _CHUNK)
    yi2d = y_idx.astype(jnp.int32).reshape(P_PAIRS // _CHUNK, _CHUNK)
    probs = _run_decode(zx, zy, xi2d, yi2d, d2_vec, b2_vec)
    return (probs.reshape(P_PAIRS, 1), z)


# speedup vs baseline: 1.1136x; 1.1136x over previous
import functools

import jax
import jax.numpy as jnp
from jax import lax
from jax.experimental import pallas as pl
from jax.experimental.pallas import tpu as pltpu
from jax.experimental.pallas import tpu_sc as plsc

N = 10000
H = 128
P_PAIRS = 65536

_BM = 400


def _dot(a, b):
    return lax.dot_general(
        a.astype(jnp.bfloat16), b.astype(jnp.bfloat16),
        (((1,), (0,)), ((), ())),
        precision=lax.Precision.DEFAULT,
        preferred_element_type=jnp.float32,
    )


def _prelude_body(x_ref, win_ref, bin_ref, whn_ref, h_ref, nb_ref):
    a = _dot(x_ref[...], win_ref[...]) + bin_ref[...]
    i = jax.nn.sigmoid(a[:, H:2 * H])
    ng = jnp.tanh(a[:, 2 * H:])
    h = (1.0 - i) * ng
    h_ref[...] = h
    nb_ref[...] = _dot(h, whn_ref[...]).astype(jnp.bfloat16)


def _gate_update(adj_ref, nb_ref, x_ref, h_ref, win_ref, bin_ref, whs_ref):
    g = _dot(adj_ref[...], nb_ref[...])
    a = _dot(x_ref[...], win_ref[...]) + bin_ref[...]
    s = _dot(h_ref[...], whs_ref[...])
    r = jax.nn.sigmoid(a[:, :H] + s[:, :H] + g[:, :H])
    i = jax.nn.sigmoid(a[:, H:2 * H] + s[:, H:2 * H] + g[:, H:2 * H])
    ng = jnp.tanh(a[:, 2 * H:] + r * (s[:, 2 * H:] + g[:, 2 * H:]))
    return (1.0 - i) * ng + i * h_ref[...]


def _step_body(adj_ref, nb_ref, x_ref, h_ref, win_ref, bin_ref, whs_ref,
               whn_ref, hout_ref, nbout_ref):
    hn = _gate_update(adj_ref, nb_ref, x_ref, h_ref, win_ref, bin_ref,
                      whs_ref)
    hout_ref[...] = hn
    nbout_ref[...] = _dot(hn, whn_ref[...]).astype(jnp.bfloat16)


def _final_body(adj_ref, nb_ref, x_ref, h_ref, win_ref, bin_ref, whs_ref,
                fc1w_ref, fc1b_ref, fc2w_ref, fc2b_ref, fc3w_ref, fc3b_ref,
                lng_ref, lnb_ref, d1x_ref, d1y_ref, d1b_ref,
                z_ref, zx_ref, zy_ref):
    hn = _gate_update(adj_ref, nb_ref, x_ref, h_ref, win_ref, bin_ref,
                      whs_ref)
    x = jnp.maximum(_dot(hn, fc1w_ref[...]) + fc1b_ref[...], 0.0)
    x = jnp.maximum(_dot(x, fc2w_ref[...]) + fc2b_ref[...], 0.0)
    x = jnp.maximum(_dot(x, fc3w_ref[...]) + fc3b_ref[...], 0.0)
    m = jnp.mean(x, axis=-1, keepdims=True)
    xc = x - m
    var = jnp.sum(xc * xc, axis=-1, keepdims=True) / (H - 1)
    z = lng_ref[...] * xc / (jnp.sqrt(var) + 1e-6) + lnb_ref[...]
    z_ref[...] = z
    zx_ref[...] = _dot(z, d1x_ref[...]) + d1b_ref[...]
    zy_ref[...] = _dot(z, d1y_ref[...])


def _row_spec(bm, cols):
    return pl.BlockSpec((bm, cols), lambda i: (i, 0))


def _full_spec(rows, cols):
    return pl.BlockSpec((rows, cols), lambda i: (0, 0))


_TC_PARAMS = pltpu.CompilerParams(
    dimension_semantics=("arbitrary",),
    vmem_limit_bytes=100 * 1024 * 1024,
)


def _run_prelude(inputs, win, bin_, whn):
    bm = 1000
    f = inputs.shape[1]
    return pl.pallas_call(
        _prelude_body,
        grid=(N // bm,),
        in_specs=[
            _row_spec(bm, f),
            _full_spec(f, 3 * H),
            _full_spec(1, 3 * H),
            _full_spec(H, 3 * H),
        ],
        out_specs=[
            _row_spec(bm, H),
            _row_spec(bm, 3 * H),
        ],
        out_shape=[
            jax.ShapeDtypeStruct((N, H), jnp.float32),
            jax.ShapeDtypeStruct((N, 3 * H), jnp.bfloat16),
        ],
        compiler_params=_TC_PARAMS,
    )(inputs, win, bin_, whn)


def _run_step(adj, nb, inputs, h, win, bin_, whs, whn):
    return pl.pallas_call(
        _step_body,
        grid=(N // _BM,),
        in_specs=[
            _row_spec(_BM, N),
            _full_spec(N, 3 * H),
            _row_spec(_BM, H),
            _row_spec(_BM, H),
            _full_spec(H, 3 * H),
            _full_spec(1, 3 * H),
            _full_spec(H, 3 * H),
            _full_spec(H, 3 * H),
        ],
        out_specs=[
            _row_spec(_BM, H),
            _row_spec(_BM, 3 * H),
        ],
        out_shape=[
            jax.ShapeDtypeStruct((N, H), jnp.float32),
            jax.ShapeDtypeStruct((N, 3 * H), jnp.bfloat16),
        ],
        compiler_params=_TC_PARAMS,
    )(adj, nb, inputs, h, win, bin_, whs, whn)


def _run_final(adj, nb, inputs, h, win, bin_, whs,
               fc1w, fc1b, fc2w, fc2b, fc3w, fc3b,
               lng, lnb, d1x, d1y, d1b):
    return pl.pallas_call(
        _final_body,
        grid=(N // _BM,),
        in_specs=[
            _row_spec(_BM, N),
            _full_spec(N, 3 * H),
            _row_spec(_BM, H),
            _row_spec(_BM, H),
            _full_spec(H, 3 * H),
            _full_spec(1, 3 * H),
            _full_spec(H, 3 * H),
            _full_spec(H, H), _full_spec(1, H),
            _full_spec(H, H), _full_spec(1, H),
            _full_spec(H, H), _full_spec(1, H),
            _full_spec(1, H), _full_spec(1, H),
            _full_spec(H, H), _full_spec(H, H), _full_spec(1, H),
        ],
        out_specs=[
            _row_spec(_BM, H),
            _row_spec(_BM, H),
            _row_spec(_BM, H),
        ],
        out_shape=[
            jax.ShapeDtypeStruct((N, H), jnp.float32),
            jax.ShapeDtypeStruct((N, H), jnp.float32),
            jax.ShapeDtypeStruct((N, H), jnp.float32),
        ],
        compiler_params=_TC_PARAMS,
    )(adj, nb, inputs, h, win, bin_, whs, fc1w, fc1b, fc2w, fc2b,
      fc3w, fc3b, lng, lnb, d1x, d1y, d1b)


_CHUNK = 128
_GROUPS = _CHUNK // 16


def _decode_body(zx_hbm, zy_hbm, xi_hbm, yi_hbm, d2_hbm, b2_hbm, out_hbm,
                 idxx_v, idxy_v, rxa, rya, rxb, ryb, d2_v, b2_v, out_v,
                 accbuf_v, semxa, semya, semxb, semyb):
    info = plsc.get_sparse_core_info()
    nc = info.num_cores
    wid = lax.axis_index("s") * nc + lax.axis_index("c")
    chunks = idxx_v.shape[0]

    pltpu.sync_copy(d2_hbm, d2_v)
    pltpu.sync_copy(b2_hbm, b2_v)
    pltpu.sync_copy(xi_hbm.at[pl.ds(wid * chunks, chunks)], idxx_v)
    pltpu.sync_copy(yi_hbm.at[pl.ds(wid * chunks, chunks)], idxy_v)

    lanes0 = lax.iota(jnp.int32, 16)

    def fire(ch, rx, ry, semx, semy):
        pltpu.async_copy(zx_hbm.at[idxx_v.at[ch]], rx, semx)
        pltpu.async_copy(zy_hbm.at[idxy_v.at[ch]], ry, semy)

    def wait(rx, ry, semx, semy):
        pltpu.make_async_copy(zx_hbm.at[idxx_v.at[0]], rx, semx).wait()
        pltpu.make_async_copy(zy_hbm.at[idxy_v.at[0]], ry, semy).wait()

    d2regs = [d2_v[pl.ds(16 * j, 16)] for j in range(H // 16)]

    def compute(ch, rx, ry):
        def group_body(g, _):
            for p in range(16):
                pr = g * 16 + p
                acc = jnp.zeros((16,), jnp.float32)
                for j in range(H // 16):
                    vx = rx[pr, pl.ds(16 * j, 16)]
                    vy = ry[pr, pl.ds(16 * j, 16)]
                    acc = acc + jnp.maximum(vx + vy, 0.0) * d2regs[j]
                accbuf_v[p, pl.ds(0, 16)] = acc
            tot = b2_v[...]
            for l in range(16):
                col = jnp.full((16,), l, jnp.int32)
                tot = tot + plsc.load_gather(accbuf_v, [lanes0, col])
            prob = 1.0 / (1.0 + jnp.exp(-tot))
            out_v[ch, pl.ds(g * 16, 16)] = prob
            return _

        lax.fori_loop(0, _GROUPS, group_body, 0, unroll=False)

    fire(0, rxa, rya, semxa, semya)

    def pair_body(i, _):
        fire(2 * i + 1, rxb, ryb, semxb, semyb)
        wait(rxa, rya, semxa, semya)
        compute(2 * i, rxa, rya)

        @pl.when(i < chunks // 2 - 1)
        def _fire_next():
            fire(2 * i + 2, rxa, rya, semxa, semya)

        wait(rxb, ryb, semxb, semyb)
        compute(2 * i + 1, rxb, ryb)
        return _

    lax.fori_loop(0, chunks // 2, pair_body, 0, unroll=False)
    pltpu.sync_copy(out_v, out_hbm.at[pl.ds(wid * chunks, chunks)])


def _run_decode(zx, zy, xi2d, yi2d, d2_vec, b2_vec):
    nw = 32
    chunks = P_PAIRS // (_CHUNK * nw)
    mesh = plsc.VectorSubcoreMesh(core_axis_name="c", subcore_axis_name="s")
    k = functools.partial(
        pl.kernel,
        out_type=jax.ShapeDtypeStruct((P_PAIRS // _CHUNK, _CHUNK), jnp.float32),
        mesh=mesh,
        compiler_params=pltpu.CompilerParams(needs_layout_passes=False),
        scratch_types=[
            pltpu.VMEM((chunks, _CHUNK), jnp.int32),
            pltpu.VMEM((chunks, _CHUNK), jnp.int32),
            pltpu.VMEM((_CHUNK, H), jnp.float32),
            pltpu.VMEM((_CHUNK, H), jnp.float32),
            pltpu.VMEM((_CHUNK, H), jnp.float32),
            pltpu.VMEM((_CHUNK, H), jnp.float32),
            pltpu.VMEM((H,), jnp.float32),
            pltpu.VMEM((16,), jnp.float32),
            pltpu.VMEM((chunks, _CHUNK), jnp.float32),
            pltpu.VMEM((16, 17), jnp.float32),
            pltpu.SemaphoreType.DMA,
            pltpu.SemaphoreType.DMA,
            pltpu.SemaphoreType.DMA,
            pltpu.SemaphoreType.DMA,
        ],
    )(_decode_body)
    return k(zx, zy, xi2d, yi2d, d2_vec, b2_vec)


def kernel(inputs, adj, x_idx, y_idx, ir_w, ir_b, ii_w, ii_b, in_w, in_b,
           hr_s, hr_n, hi_s, hi_n, hh_s, hh_n,
           fc1_w, fc1_b, fc2_w, fc2_b, fc3_w, fc3_b,
           ln_g, ln_b, d1_w, d1_b, d2_w, d2_b):
    win = jnp.concatenate([ir_w, ii_w, in_w], axis=1)
    bin_ = jnp.concatenate([ir_b, ii_b, in_b]).reshape(1, 3 * H)
    whs = jnp.concatenate([hr_s, hi_s, hh_s], axis=1)
    whn = jnp.concatenate([hr_n, hi_n, hh_n], axis=1)

    h1, nb1 = _run_prelude(inputs, win, bin_, whn)
    h2, nb2 = _run_step(adj, nb1, inputs, h1, win, bin_, whs, whn)
    z, zx, zy = _run_final(
        adj, nb2, inputs, h2, win, bin_, whs,
        fc1_w, fc1_b.reshape(1, H), fc2_w, fc2_b.reshape(1, H),
        fc3_w, fc3_b.reshape(1, H),
        ln_g.reshape(1, H), ln_b.reshape(1, H),
        d1_w[:H], d1_w[H:], d1_b.reshape(1, H))

    d2_vec = d2_w.reshape(H)
    b2_vec = jnp.full((16,), d2_b[0], jnp.float32)
    xi2d = x_idx.astype(jnp.int32).reshape(P_PAIRS // _CHUNK, _CHUNK)
    yi2d = y_idx.astype(jnp.int32).reshape(P_PAIRS // _CHUNK, _CHUNK)
    probs = _run_decode(zx, zy, xi2d, yi2d, d2_vec, b2_vec)
    return (probs.reshape(P_PAIRS, 1), z)
